# baseline (speedup 1.0000x reference)
import jax
import jax.numpy as jnp
from jax import lax
from jax.experimental import pallas as pl


def _iou_row(box, gt_boxes):
    x1 = jnp.maximum(box[0], gt_boxes[:, 0])
    y1 = jnp.maximum(box[1], gt_boxes[:, 1])
    x2 = jnp.minimum(box[2], gt_boxes[:, 2])
    y2 = jnp.minimum(box[3], gt_boxes[:, 3])
    inter = jnp.clip(x2 - x1, 0.0) * jnp.clip(y2 - y1, 0.0)
    area_p = (box[2] - box[0]) * (box[3] - box[1])
    area_g = (gt_boxes[:, 2] - gt_boxes[:, 0]) * (gt_boxes[:, 3] - gt_boxes[:, 1])
    return inter / (area_p + area_g - inter + 1e-9)


def kernel(pred_scores, pred_boxes, pred_labels, gt_boxes):
    order = jnp.argsort(-pred_scores)
    boxes_sorted = pred_boxes[order]
    M = gt_boxes.shape[0]

    def step(matched, box):
        iou = _iou_row(box, gt_boxes)
        iou = jnp.where(matched > 0.5, -1.0, iou)
        best = jnp.argmax(iou)
        best_iou = iou[best]
        is_tp = (best_iou >= 0.5).astype(jnp.float32)
        matched = matched.at[best].max(is_tp)
        return matched, is_tp

    _, tp = lax.scan(step, jnp.zeros((M,), jnp.float32), boxes_sorted)
    fp = 1.0 - tp
    cum_tp = jnp.cumsum(tp)
    cum_fp = jnp.cumsum(fp)
    recall = cum_tp / jnp.maximum(jnp.float32(M), 1.0)
    precision = cum_tp / jnp.maximum(cum_tp + cum_fp, 1e-9)
    thresholds = jnp.linspace(0.0, 1.0, 11)

    def prec_at(t):
        return jnp.max(jnp.where(recall >= t, precision, 0.0))

    return jnp.mean(jax.vmap(prec_at)(thresholds))


# speedup vs baseline: 40.2300x; 40.2300x over previous
import functools

import jax
import jax.numpy as jnp
from jax import lax
from jax.experimental import pallas as pl
from jax.experimental.pallas import tpu as pltpu
from jax.experimental.pallas import tpu_sc as plsc

K = 4
TN = 32
TR = 32
IOU_T = 0.5


def _topk_body(px1, py1, px2, py2, gx1, gy1, gx2, gy2, out):
    mp = gx1.shape[1]
    x1 = jnp.maximum(px1[...], gx1[...])
    y1 = jnp.maximum(py1[...], gy1[...])
    x2 = jnp.minimum(px2[...], gx2[...])
    y2 = jnp.minimum(py2[...], gy2[...])
    inter = jnp.maximum(x2 - x1, 0.0) * jnp.maximum(y2 - y1, 0.0)
    area_p = (px2[...] - px1[...]) * (py2[...] - py1[...])
    area_g = (gx2[...] - gx1[...]) * (gy2[...] - gy1[...])
    iou = inter / (area_p + area_g - inter + 1e-9)
    lanes = lax.broadcasted_iota(jnp.int32, (px1.shape[0], mp), 1)
    sent = jnp.int32(mp)
    for k in range(K):
        m = jnp.max(iou, axis=1, keepdims=True)
        idx = jnp.min(jnp.where(iou == m, lanes, jnp.int32(mp)), axis=1,
                      keepdims=True)
        out[:, k : k + 1] = jnp.where(m >= IOU_T, idx, sent)
        iou = jnp.where(lanes == idx, -1.0, iou)


def _rank_body(srow, scol, out):
    tr = srow.shape[0]
    np_ = scol.shape[1]
    pid = pl.program_id(0)
    row_id = pid * tr + lax.broadcasted_iota(jnp.int32, (tr, 1), 0)
    col_id = lax.broadcasted_iota(jnp.int32, (1, np_), 1)
    sr = srow[...]
    sc = scol[...]
    gt = sc > sr
    tie = jnp.logical_and(sc == sr, col_id < row_id)
    cnt = jnp.where(jnp.logical_or(gt, tie), jnp.int32(1), jnp.int32(0))
    out[...] = jnp.sum(cnt, axis=1, keepdims=True)


def _make_sc_match(n, m, nk):
    mesh = plsc.VectorSubcoreMesh(
        core_axis_name="c", subcore_axis_name="s", num_cores=2,
        num_subcores=16)
    m_pad = ((m + 16) // 16) * 16
    chunk = 2000

    @functools.partial(
        pl.kernel,
        out_type=jax.ShapeDtypeStruct((16,), jnp.float32),
        mesh=mesh,
        compiler_params=pltpu.CompilerParams(needs_layout_passes=False),
        scratch_types=[
            pltpu.VMEM((n,), jnp.int32),
            pltpu.VMEM((nk,), jnp.int32),
            pltpu.VMEM((m_pad,), jnp.int32),
            pltpu.VMEM((m_pad,), jnp.int32),
            pltpu.VMEM((chunk,), jnp.int32),
            pltpu.VMEM((16,), jnp.float32),
            pltpu.VMEM((16,), jnp.float32),
        ],
    )
    def sc_match(rank_hbm, cand_hbm, th_hbm, out_hbm, inv_v, cand_v,
                 matched_v, ptp_v, rbuf_v, th_v, obuf_v):
        wid = lax.axis_index("s") * 2 + lax.axis_index("c")

        @pl.when(wid == 0)
        def _():
            pltpu.sync_copy(cand_hbm, cand_v)
            pltpu.sync_copy(th_hbm, th_v)

            def zero_body(i, _):
                matched_v[pl.ds(i * 16, 16)] = jnp.zeros((16,), jnp.int32)
                return 0
            lax.fori_loop(0, m_pad // 16, zero_body, 0)

            def inv_chunk(c, _):
                pltpu.sync_copy(rank_hbm.at[pl.ds(c * chunk, chunk)], rbuf_v)

                def inv_body(j, _):
                    r16 = rbuf_v[pl.ds(j * 16, 16)]
                    ids = (c * chunk + j * 16
                           + lax.iota(jnp.int32, 16)).astype(jnp.int32)
                    plsc.store_scatter(inv_v, [r16], ids)
                    return 0
                lax.fori_loop(0, chunk // 16, inv_body, 0)
                return 0
            lax.fori_loop(0, n // chunk, inv_chunk, 0)

            iota16 = lax.iota(jnp.int32, 16)
            lane0 = iota16 == 0

            def step(i, t):
                pvec = plsc.load_gather(inv_v, [jnp.full((16,), i, jnp.int32)])
                p = pvec[0]
                cidx = jnp.where(iota16 < K, p * K + iota16, 0)
                cands = plsc.load_gather(cand_v, [cidx])
                cands = jnp.where(iota16 < K, cands, jnp.int32(m))
                valid = cands < m
                mstat = plsc.load_gather(
                    matched_v, [jnp.where(valid, cands, 0)])
                fv = jnp.logical_and(valid, mstat == 0)
                cnt = plsc.all_reduce_population_count(fv)
                ffs = plsc.all_reduce_ffs(fv)
                gsel = jnp.where(iota16 == ffs, cands, 0)
                g = jnp.max(gsel)
                hit = jnp.max(cnt) > 0

                @pl.when(hit)
                def _():
                    plsc.store_scatter(
                        matched_v, [jnp.full((16,), g, jnp.int32)],
                        jnp.ones((16,), jnp.int32), mask=lane0)
                    plsc.store_scatter(
                        ptp_v, [jnp.full((16,), t + 1, jnp.int32)],
                        jnp.full((16,), i + 1, jnp.int32), mask=lane0)

                return t + jnp.where(hit, jnp.int32(1), jnp.int32(0))

            total_tp = lax.fori_loop(0, n, step, jnp.int32(0))

            th = th_v[...]
            mfv = jnp.full((16,), jnp.float32(m))

            def ap_body(jj, carry):
                rmax, ans = carry
                j = total_tp - jj
                jv = jnp.full((16,), j, jnp.int32)
                pjv = plsc.load_gather(ptp_v, [jv])
                pr = jv.astype(jnp.float32) / pjv.astype(jnp.float32)
                rmax = jnp.maximum(rmax, pr)
                recall = jv.astype(jnp.float32) / mfv
                mask = recall >= th
                ans = jnp.where(mask, rmax, ans)
                return rmax, ans

            _, ans = lax.fori_loop(
                0, total_tp, ap_body,
                (jnp.zeros((16,), jnp.float32),
                 jnp.zeros((16,), jnp.float32)))
            w = jnp.where(iota16 < 11, jnp.float32(1.0), jnp.float32(0.0))
            apsum = jnp.sum(ans * w, axis=0)
            apv = jnp.full((16,), apsum) / jnp.full((16,), jnp.float32(11.0))
            obuf_v[...] = apv
            pltpu.sync_copy(obuf_v, out_hbm)

    return sc_match


def kernel(pred_scores, pred_boxes, pred_labels, gt_boxes):
    n = pred_boxes.shape[0]
    m = gt_boxes.shape[0]
    mp = ((m + 127) // 128) * 128
    np_ = ((n + 127) // 128) * 128

    pxy = [pred_boxes[:, c : c + 1] for c in range(4)]
    gpad = jnp.full((mp - m, 4), 3e9, jnp.float32)
    gt_p = jnp.concatenate([gt_boxes, gpad], axis=0)
    gxy = [gt_p[:, c].reshape(1, mp) for c in range(4)]

    cand = pl.pallas_call(
        _topk_body,
        grid=(n // TN,),
        in_specs=[pl.BlockSpec((TN, 1), lambda i: (i, 0))] * 4
        + [pl.BlockSpec((1, mp), lambda i: (0, 0))] * 4,
        out_specs=pl.BlockSpec((TN, K), lambda i: (i, 0)),
        out_shape=jax.ShapeDtypeStruct((n, K), jnp.int32),
    )(*pxy, *gxy)

    s_col = jnp.concatenate(
        [pred_scores, jnp.full((np_ - n,), -1.0, jnp.float32)]).reshape(1, np_)
    rank = pl.pallas_call(
        _rank_body,
        grid=(n // TR,),
        in_specs=[
            pl.BlockSpec((TR, 1), lambda i: (i, 0)),
            pl.BlockSpec((1, np_), lambda i: (0, 0)),
        ],
        out_specs=pl.BlockSpec((TR, 1), lambda i: (i, 0)),
        out_shape=jax.ShapeDtypeStruct((n, 1), jnp.int32),
    )(pred_scores.reshape(n, 1), s_col)

    th = jnp.concatenate(
        [jnp.linspace(0.0, 1.0, 11), jnp.full((5,), 2.0)]).astype(jnp.float32)

    sc = _make_sc_match(n, m, n * K)
    ap16 = sc(rank.reshape(n), cand.reshape(n * K), th)
    return ap16[0]


# speedup vs baseline: 58.4098x; 1.4519x over previous
import functools

import jax
import jax.numpy as jnp
from jax import lax
from jax.experimental import pallas as pl
from jax.experimental.pallas import tpu as pltpu
from jax.experimental.pallas import tpu_sc as plsc

K = 4
TN = 32
TR = 32
IOU_T = 0.5


def _topk_body(px1, py1, px2, py2, gx1, gy1, gx2, gy2, out):
    mp = gx1.shape[1]
    x1 = jnp.maximum(px1[...], gx1[...])
    y1 = jnp.maximum(py1[...], gy1[...])
    x2 = jnp.minimum(px2[...], gx2[...])
    y2 = jnp.minimum(py2[...], gy2[...])
    inter = jnp.maximum(x2 - x1, 0.0) * jnp.maximum(y2 - y1, 0.0)
    area_p = (px2[...] - px1[...]) * (py2[...] - py1[...])
    area_g = (gx2[...] - gx1[...]) * (gy2[...] - gy1[...])
    iou = inter / (area_p + area_g - inter + 1e-9)
    lanes = lax.broadcasted_iota(jnp.int32, (px1.shape[0], mp), 1)
    sent = jnp.int32(mp)
    for k in range(K):
        m = jnp.max(iou, axis=1, keepdims=True)
        idx = jnp.min(jnp.where(iou == m, lanes, jnp.int32(mp)), axis=1,
                      keepdims=True)
        out[:, k : k + 1] = jnp.where(m >= IOU_T, idx, sent)
        iou = jnp.where(lanes == idx, -1.0, iou)


def _rank_body(srow, scol, out):
    tr = srow.shape[0]
    np_ = scol.shape[1]
    pid = pl.program_id(0)
    row_id = pid * tr + lax.broadcasted_iota(jnp.int32, (tr, 1), 0)
    col_id = lax.broadcasted_iota(jnp.int32, (1, np_), 1)
    sr = srow[...]
    sc = scol[...]
    gt = sc > sr
    tie = jnp.logical_and(sc == sr, col_id < row_id)
    cnt = jnp.where(jnp.logical_or(gt, tie), jnp.int32(1), jnp.int32(0))
    out[...] = jnp.sum(cnt, axis=1, keepdims=True)


def _make_sc_match(n, m, nk):
    mesh = plsc.VectorSubcoreMesh(
        core_axis_name="c", subcore_axis_name="s", num_cores=2,
        num_subcores=16)
    m_pad = ((m + 16) // 16) * 16
    chunk = 2000

    @functools.partial(
        pl.kernel,
        out_type=jax.ShapeDtypeStruct((16,), jnp.float32),
        mesh=mesh,
        compiler_params=pltpu.CompilerParams(needs_layout_passes=False),
        scratch_types=[
            pltpu.VMEM((n + 16,), jnp.int32),
            pltpu.VMEM((nk,), jnp.int32),
            pltpu.VMEM((m_pad,), jnp.int32),
            pltpu.VMEM((m_pad,), jnp.int32),
            pltpu.VMEM((chunk,), jnp.int32),
            pltpu.VMEM((16,), jnp.float32),
            pltpu.VMEM((16,), jnp.float32),
        ],
    )
    def sc_match(rank_hbm, cand_hbm, th_hbm, out_hbm, work_v, cand_v,
                 matched_v, ptp_v, rbuf_v, th_v, obuf_v):
        wid = lax.axis_index("s") * 2 + lax.axis_index("c")

        @pl.when(wid == 0)
        def _():
            pltpu.sync_copy(cand_hbm, cand_v)
            pltpu.sync_copy(th_hbm, th_v)

            def zero_body(i, _):
                matched_v[pl.ds(i * 16, 16)] = jnp.zeros((16,), jnp.int32)
                return 0
            lax.fori_loop(0, m_pad // 16, zero_body, 0)

            iota16 = lax.iota(jnp.int32, 16)
            lane0 = iota16 == 0
            sent = jnp.int32(32767)

            def inv_chunk(c, _):
                pltpu.sync_copy(rank_hbm.at[pl.ds(c * chunk, chunk)], rbuf_v)

                def inv_body(j, _):
                    r16 = rbuf_v[pl.ds(j * 16, 16)]
                    ids = c * chunk + j * 16 + iota16
                    c16 = plsc.load_gather(cand_v, [ids * K])
                    fl = c16 < m
                    plsc.store_scatter(work_v, [r16],
                                       jnp.where(fl, ids, sent))
                    return 0
                lax.fori_loop(0, chunk // 16, inv_body, 0)
                return 0
            lax.fori_loop(0, n // chunk, inv_chunk, 0)

            def comp_body(j, nc):
                v16 = work_v[pl.ds(j * 16, 16)]
                fl = v16 < sent
                incl = plsc.cumsum(jnp.where(fl, jnp.int32(1), jnp.int32(0)))
                pos = jnp.where(fl, nc + incl - 1, 0)
                packed = (j * 16 + iota16) * 32768 + v16
                plsc.store_scatter(work_v, [pos], packed, mask=fl)
                return nc + incl[15]
            nc = lax.fori_loop(0, n // 16, comp_body, jnp.int32(0))

            def step(ii, t):
                pk = plsc.load_gather(work_v, [jnp.full((16,), ii, jnp.int32)])
                packed = pk[0]
                p = jnp.bitwise_and(packed, jnp.int32(32767))
                gpos = lax.shift_right_logical(packed, jnp.int32(15))
                cidx = jnp.where(iota16 < K, p * K + iota16, 0)
                cands = plsc.load_gather(cand_v, [cidx])
                cands = jnp.where(iota16 < K, cands, jnp.int32(m))
                valid = cands < m
                mstat = plsc.load_gather(
                    matched_v, [jnp.where(valid, cands, 0)])
                fv = jnp.logical_and(valid, mstat == 0)
                cnt = plsc.all_reduce_population_count(fv)
                ffs = plsc.all_reduce_ffs(fv)
                gsel = jnp.where(iota16 == ffs, cands, 0)
                g = jnp.max(gsel)
                hit = jnp.max(cnt) > 0

                @pl.when(hit)
                def _():
                    plsc.store_scatter(
                        matched_v, [jnp.full((16,), g, jnp.int32)],
                        jnp.ones((16,), jnp.int32), mask=lane0)
                    plsc.store_scatter(
                        ptp_v, [jnp.full((16,), t + 1, jnp.int32)],
                        jnp.full((16,), gpos + 1, jnp.int32), mask=lane0)

                return t + jnp.where(hit, jnp.int32(1), jnp.int32(0))

            total_tp = lax.fori_loop(0, nc, step, jnp.int32(0))

            th = th_v[...]
            mfv = jnp.full((16,), jnp.float32(m))

            def ap_body(jj, carry):
                rmax, ans = carry
                j = total_tp - jj
                jv = jnp.full((16,), j, jnp.int32)
                pjv = plsc.load_gather(ptp_v, [jv])
                pr = jv.astype(jnp.float32) / pjv.astype(jnp.float32)
                rmax = jnp.maximum(rmax, pr)
                recall = jv.astype(jnp.float32) / mfv
                mask = recall >= th
                ans = jnp.where(mask, rmax, ans)
                return rmax, ans

            _, ans = lax.fori_loop(
                0, total_tp, ap_body,
                (jnp.zeros((16,), jnp.float32),
                 jnp.zeros((16,), jnp.float32)))
            w = jnp.where(iota16 < 11, jnp.float32(1.0), jnp.float32(0.0))
            apsum = jnp.sum(ans * w, axis=0)
            apv = jnp.full((16,), apsum) / jnp.full((16,), jnp.float32(11.0))
            obuf_v[...] = apv
            pltpu.sync_copy(obuf_v, out_hbm)

    return sc_match


def kernel(pred_scores, pred_boxes, pred_labels, gt_boxes):
    n = pred_boxes.shape[0]
    m = gt_boxes.shape[0]
    mp = ((m + 127) // 128) * 128
    np_ = ((n + 127) // 128) * 128

    pxy = [pred_boxes[:, c : c + 1] for c in range(4)]
    gpad = jnp.full((mp - m, 4), 3e9, jnp.float32)
    gt_p = jnp.concatenate([gt_boxes, gpad], axis=0)
    gxy = [gt_p[:, c].reshape(1, mp) for c in range(4)]

    cand = pl.pallas_call(
        _topk_body,
        grid=(n // TN,),
        in_specs=[pl.BlockSpec((TN, 1), lambda i: (i, 0))] * 4
        + [pl.BlockSpec((1, mp), lambda i: (0, 0))] * 4,
        out_specs=pl.BlockSpec((TN, K), lambda i: (i, 0)),
        out_shape=jax.ShapeDtypeStruct((n, K), jnp.int32),
    )(*pxy, *gxy)

    s_col = jnp.concatenate(
        [pred_scores, jnp.full((np_ - n,), -1.0, jnp.float32)]).reshape(1, np_)
    rank = pl.pallas_call(
        _rank_body,
        grid=(n // TR,),
        in_specs=[
            pl.BlockSpec((TR, 1), lambda i: (i, 0)),
            pl.BlockSpec((1, np_), lambda i: (0, 0)),
        ],
        out_specs=pl.BlockSpec((TR, 1), lambda i: (i, 0)),
        out_shape=jax.ShapeDtypeStruct((n, 1), jnp.int32),
    )(pred_scores.reshape(n, 1), s_col)

    th = jnp.concatenate(
        [jnp.linspace(0.0, 1.0, 11), jnp.full((5,), 2.0)]).astype(jnp.float32)

    sc = _make_sc_match(n, m, n * K)
    ap16 = sc(rank.reshape(n), cand.reshape(n * K), th)
    return ap16[0]
